# Initial kernel scaffold; baseline (speedup 1.0000x reference)
import jax
import jax.numpy as jnp
from jax.experimental import pallas as pl


def kernel(grid, locs):
    raise NotImplementedError("write your pallas kernel here")


# speedup vs baseline: 2.7834x; 2.7834x over previous
import functools

import jax
import jax.numpy as jnp
from jax import lax
from jax.experimental import pallas as pl
from jax.experimental.pallas import tpu as pltpu
from jax.experimental.pallas import tpu_sc as plsc

GRID_LOWER = (0.0, 0.0, 0.0)
GRID_STEPS = (0.015625, 0.015625, 0.015625)

NC, NS, L = 2, 16, 16
NW = NC * NS
CHUNK = 128
GROUPS = CHUNK // L


def _bcast_lane(v, lane):
    idx = jnp.full((L,), lane, jnp.int32)
    dn = lax.GatherDimensionNumbers(
        offset_dims=(), collapsed_slice_dims=(0,), start_index_map=(0,))
    return lax.gather(v, idx[:, None], dn, (1,),
                      mode=lax.GatherScatterMode.PROMISE_IN_BOUNDS)


def _axis_coords(cv, hi):
    t = (cv + 64.0).astype(jnp.int32)
    i0 = t - 64
    f = cv - i0.astype(jnp.float32)
    i0c = jnp.clip(i0, 0, hi)
    i1c = jnp.clip(i0 + 1, 0, hi)
    return i0c, i1c, f


def _make_sc_call(XYZ, NPB, npt, x_dim, y_dim, z_dim, C):
    nchunks = npt // CHUNK
    mesh = plsc.VectorSubcoreMesh(
        core_axis_name="c", subcore_axis_name="s",
        num_cores=NC, num_subcores=NS)
    total = NW * npt
    inv_step = 1.0 / GRID_STEPS[0]

    @functools.partial(
        pl.kernel,
        out_type=jax.ShapeDtypeStruct((total, C), jnp.float32),
        mesh=mesh,
        scratch_types=[
            pltpu.VMEM((npt,), jnp.float32),
            pltpu.VMEM((npt,), jnp.float32),
            pltpu.VMEM((npt,), jnp.float32),
            pltpu.VMEM((8, CHUNK), jnp.int32),
            pltpu.VMEM((8, CHUNK), jnp.int32),
            pltpu.VMEM((8, CHUNK, C), jnp.float32),
            pltpu.VMEM((8, CHUNK, C), jnp.float32),
            pltpu.VMEM((3, CHUNK), jnp.float32),
            pltpu.VMEM((3, CHUNK), jnp.float32),
            pltpu.VMEM((CHUNK, C), jnp.float32),
            pltpu.VMEM((CHUNK, C), jnp.float32),
            pltpu.SemaphoreType.DMA,
            pltpu.SemaphoreType.DMA,
        ],
        compiler_params=pltpu.CompilerParams(use_tc_tiling_on_sc=False),
    )
    def sc_call(gflat, xs, ys, zs, out,
                xv, yv, zv, idxA, idxB, rowsA, rowsB, frA, frB,
                outA, outB, semA, semB):
        cid = lax.axis_index("c")
        sid = lax.axis_index("s")
        wid = sid * NC + cid
        base = wid * npt
        boff = (base // NPB) * XYZ

        pltpu.sync_copy(xs.at[pl.ds(base, npt)], xv)
        pltpu.sync_copy(ys.at[pl.ds(base, npt)], yv)
        pltpu.sync_copy(zs.at[pl.ds(base, npt)], zv)

        def stage(c, idx_r, fr_r):
            off = c * CHUNK
            for g in range(GROUPS):
                s = off + g * L
                cx = xv[pl.ds(s, L)] * inv_step - 0.5
                cy = yv[pl.ds(s, L)] * inv_step - 0.5
                cz = zv[pl.ds(s, L)] * inv_step - 0.5
                x0, x1, fx = _axis_coords(cx, x_dim - 1)
                y0, y1, fy = _axis_coords(cy, y_dim - 1)
                z0, z1, fz = _axis_coords(cz, z_dim - 1)
                xb = (x0 * (y_dim * z_dim) + boff, x1 * (y_dim * z_dim) + boff)
                yb = (y0 * z_dim, y1 * z_dim)
                zb = (z0, z1)
                fr_r[0, pl.ds(g * L, L)] = fx
                fr_r[1, pl.ds(g * L, L)] = fy
                fr_r[2, pl.ds(g * L, L)] = fz
                for dx in (0, 1):
                    xyb = (xb[dx] + yb[0], xb[dx] + yb[1])
                    for dy in (0, 1):
                        for dz in (0, 1):
                            k = dx * 4 + dy * 2 + dz
                            idx_r[k, pl.ds(g * L, L)] = xyb[dy] + zb[dz]

        def fire(idx_r, rows_r, sem):
            for k in range(8):
                pltpu.async_copy(gflat.at[idx_r.at[k]], rows_r.at[k], sem)

        def drain(idx_r, rows_r, sem):
            for k in range(8):
                pltpu.make_async_copy(gflat.at[idx_r.at[k]], rows_r.at[k],
                                      sem).wait()

        def accum(c, rows_r, fr_r, ob):
            def pbody(p, _):
                gb = (p // L) * L
                lane = p - gb
                bx = _bcast_lane(fr_r[0, pl.ds(gb, L)], lane)
                by = _bcast_lane(fr_r[1, pl.ds(gb, L)], lane)
                bz = _bcast_lane(fr_r[2, pl.ds(gb, L)], lane)
                wx = (1.0 - bx, bx)
                wyz = ((1.0 - by) * (1.0 - bz), (1.0 - by) * bz,
                       by * (1.0 - bz), by * bz)
                w0 = wx[0] * wyz[0]
                a0 = rows_r[0, p, pl.ds(0, L)] * w0
                a1 = rows_r[0, p, pl.ds(L, L)] * w0
                for k in range(1, 8):
                    wk = wx[k // 4] * wyz[k % 4]
                    a0 = a0 + rows_r[k, p, pl.ds(0, L)] * wk
                    a1 = a1 + rows_r[k, p, pl.ds(L, L)] * wk
                ob[p, pl.ds(0, L)] = a0
                ob[p, pl.ds(L, L)] = a1
                return _

            lax.fori_loop(0, CHUNK, pbody, 0, unroll=2)
            pltpu.sync_copy(ob, out.at[pl.ds(base + c * CHUNK, CHUNK)])

        stage(0, idxA, frA)
        fire(idxA, rowsA, semA)

        def pair(i, _):
            c0 = i * 2
            stage(c0 + 1, idxB, frB)
            fire(idxB, rowsB, semB)
            drain(idxA, rowsA, semA)
            accum(c0, rowsA, frA, outA)

            @pl.when(i + 1 < nchunks // 2)
            def _fire_next():
                stage(c0 + 2, idxA, frA)
                fire(idxA, rowsA, semA)

            drain(idxB, rowsB, semB)
            accum(c0 + 1, rowsB, frB, outB)
            return _

        lax.fori_loop(0, nchunks // 2, pair, 0)

    return sc_call


def kernel(grid, locs):
    B, X, Y, Z, C = grid.shape
    N = locs.shape[1]
    NPB = -(-N // (NW * CHUNK * 2)) * (NW * CHUNK * 2)
    npt = (B * NPB) // NW

    gflat = grid.reshape(B * X * Y * Z, C)
    coord = locs.astype(jnp.float32)
    pad = NPB - N
    locs_p = jnp.pad(coord, ((0, 0), (0, pad), (0, 0)))
    flat = locs_p.reshape(B * NPB, 3)
    xs = flat[:, 0]
    ys = flat[:, 1]
    zs = flat[:, 2]

    sc_call = _make_sc_call(X * Y * Z, NPB, npt, X, Y, Z, C)
    out_p = sc_call(gflat, xs, ys, zs)
    return out_p.reshape(B, NPB, C)[:, :N, :]
